# baseline (speedup 1.0000x reference)
import jax
import jax.numpy as jnp
from jax import lax
from jax.experimental import pallas as pl
from jax.experimental.pallas import tpu as pltpu
from jax.experimental.pallas import tpu_sc as plsc

B, N, C, K = 8, 2048, 256, 1024
BLK = 128
NW = 32
RPW = (B * K) // NW


def _topk_body(nodes_ref, w_ref, b_ref, idx_ref):
    f32 = jnp.float32
    nodes = nodes_ref[0]
    w = w_ref[...]
    bias = b_ref[0, 0]
    a_col = jnp.dot(nodes, w, preferred_element_type=f32) + bias

    NB = N // BLK
    a_row = jnp.transpose(a_col)
    acs = [a_col[ib * BLK:(ib + 1) * BLK, :] for ib in range(NB)]
    rows = [a_row[:, ib * BLK:(ib + 1) * BLK] for ib in range(NB)]

    ones_col = jnp.ones((BLK, 1), f32)
    upper = (lax.broadcasted_iota(jnp.int32, (BLK, BLK), 1) >
             lax.broadcasted_iota(jnp.int32, (BLK, BLK), 0))
    T = []
    U_row = jnp.zeros((1, N), f32)
    for ib in range(NB):
        ai = acs[ib]
        dF = jnp.where((rows[ib] > ai) & upper, f32(1), f32(0))
        t = lax.dot_general(dF, ones_col, (((1,), (0,)), ((), ())),
                            preferred_element_type=f32)
        u = lax.dot_general(ones_col, dF, (((0,), (0,)), ((), ())),
                            preferred_element_type=f32)
        parts = [u]
        if ib + 1 < NB:
            ar = a_row[:, (ib + 1) * BLK:]
            wu = N - (ib + 1) * BLK
            xF = jnp.where(ar > ai, f32(1), f32(0))
            t = t + lax.dot_general(xF, jnp.ones((wu, 1), f32),
                                    (((1,), (0,)), ((), ())),
                                    preferred_element_type=f32)
            parts.append(lax.dot_general(ones_col, xF, (((0,), (0,)), ((), ())),
                                         preferred_element_type=f32))
        if ib > 0:
            parts.insert(0, jnp.zeros((1, ib * BLK), f32))
        U_row = U_row + jnp.concatenate(parts, axis=1)
        T.append(t)

    U_col = jnp.transpose(U_row)
    iif = lax.broadcasted_iota(jnp.int32, (N, 1), 0).astype(f32)
    rank_full = jnp.concatenate(T, axis=0) + iif - U_col

    p_rowF = lax.broadcasted_iota(jnp.int32, (1, K), 1).astype(f32)
    jidsF = lax.broadcasted_iota(jnp.int32, (N, 1), 0).astype(f32)
    picked = jnp.where(rank_full == p_rowF, jidsF, f32(0))
    idxF = jnp.sum(picked, axis=0, keepdims=True)
    idx_ref[0] = idxF.astype(jnp.int32)


_topk = pl.pallas_call(
    _topk_body,
    grid=(B,),
    in_specs=[pl.BlockSpec((1, N, C), lambda i: (i, 0, 0)),
              pl.BlockSpec((C, 1), lambda i: (0, 0)),
              pl.BlockSpec((1, 1), lambda i: (0, 0))],
    out_specs=pl.BlockSpec((1, 1, K), lambda i: (i, 0, 0)),
    out_shape=jax.ShapeDtypeStruct((B, 1, K), jnp.int32),
)


CH = 32
NCH = RPW // CH


def _sc_gather_body(idx_hbm, nodes_hbm, adj_hbm, nodes_out, adj_out,
                    idxv, gidx, nbuf0, nbuf1, abuf0, abuf1,
                    nsem0, nsem1, asem0, asem1,
                    nwsem0, nwsem1, awsem0, awsem1):
    wid = lax.axis_index("s") * 2 + lax.axis_index("c")
    base = wid * RPW
    pltpu.sync_copy(idx_hbm.at[pl.ds(base, RPW)], idxv)
    g = wid // (K // RPW)
    per = CH // 16
    for i in range(RPW // 16):
        v = idxv[pl.ds(i * 16, 16)]
        gidx[i // per, pl.ds((i % per) * 16, 16)] = v + g * N
    abufs = (abuf0, abuf1)
    asems = (asem0, asem1)
    nbufs = (nbuf0, nbuf1)
    nsems = (nsem0, nsem1)
    awsems = (awsem0, awsem1)
    nwsems = (nwsem0, nwsem1)
    acps = {0: pltpu.async_copy(adj_hbm.at[gidx.at[0], pl.ds(0, K)],
                                abuf0, asem0)}
    ncps = {0: pltpu.async_copy(nodes_hbm.at[gidx.at[0]], nbuf0, nsem0)}
    aw, nw = {}, {}
    for c in range(NCH):
        ncps[c].wait()
        nw[c] = pltpu.async_copy(nbufs[c % 2],
                                 nodes_out.at[pl.ds(base + c * CH, CH)],
                                 nwsems[c % 2])
        acps[c].wait()
        aw[c] = pltpu.async_copy(abufs[c % 2],
                                 adj_out.at[pl.ds(base + c * CH, CH)],
                                 awsems[c % 2])
        if c + 1 < NCH:
            if c >= 1:
                nw[c - 1].wait()
                aw[c - 1].wait()
            acps[c + 1] = pltpu.async_copy(
                adj_hbm.at[gidx.at[c + 1], pl.ds(0, K)],
                abufs[(c + 1) % 2], asems[(c + 1) % 2])
            ncps[c + 1] = pltpu.async_copy(
                nodes_hbm.at[gidx.at[c + 1]],
                nbufs[(c + 1) % 2], nsems[(c + 1) % 2])
    nw[NCH - 2].wait()
    aw[NCH - 2].wait()
    nw[NCH - 1].wait()
    aw[NCH - 1].wait()


def _sc_gather(idx_flat, nodes_flat, adj_flat):
    mesh = plsc.VectorSubcoreMesh(core_axis_name="c", subcore_axis_name="s")
    kern = pl.kernel(
        _sc_gather_body,
        out_type=[jax.ShapeDtypeStruct((B * K, C), jnp.float32),
                  jax.ShapeDtypeStruct((B * K, K), jnp.float32)],
        mesh=mesh,
        scratch_types=[pltpu.VMEM((RPW,), jnp.int32),
                       pltpu.VMEM((NCH, CH), jnp.int32),
                       pltpu.VMEM((CH, C), jnp.float32),
                       pltpu.VMEM((CH, C), jnp.float32),
                       pltpu.VMEM((CH, K), jnp.float32),
                       pltpu.VMEM((CH, K), jnp.float32),
                       pltpu.SemaphoreType.DMA,
                       pltpu.SemaphoreType.DMA,
                       pltpu.SemaphoreType.DMA,
                       pltpu.SemaphoreType.DMA,
                       pltpu.SemaphoreType.DMA,
                       pltpu.SemaphoreType.DMA,
                       pltpu.SemaphoreType.DMA,
                       pltpu.SemaphoreType.DMA],
    )
    return kern(idx_flat, nodes_flat, adj_flat)


def kernel(nodes, adj_mat, W, b):
    idx3 = _topk(nodes, W, b.reshape(1, 1))
    idx_flat = idx3.reshape(B * K)
    nodes_flat = nodes.reshape(B * N, C)
    adj_flat = adj_mat.reshape(B * N, 2 * K)
    nodes_out, adj_out = _sc_gather(idx_flat, nodes_flat, adj_flat)
    return nodes_out.reshape(B, K, C), adj_out.reshape(B, K, K)


# speedup vs baseline: 1.0185x; 1.0185x over previous
import jax
import jax.numpy as jnp
from jax import lax
from jax.experimental import pallas as pl
from jax.experimental.pallas import tpu as pltpu
from jax.experimental.pallas import tpu_sc as plsc

B, N, C, K = 8, 2048, 256, 1024
BLK = 128
NW = 32
RPW = (B * K) // NW


def _topk_body(nodes_ref, w_ref, b_ref, idx_ref):
    f32 = jnp.float32
    nodes = nodes_ref[0]
    w = w_ref[...]
    bias = b_ref[0, 0]
    a_col = jnp.dot(nodes, w, preferred_element_type=f32) + bias

    NB = N // BLK
    a_row = jnp.transpose(a_col)
    acs = [a_col[ib * BLK:(ib + 1) * BLK, :] for ib in range(NB)]
    rows = [a_row[:, ib * BLK:(ib + 1) * BLK] for ib in range(NB)]

    ones_col = jnp.ones((BLK, 1), f32)
    upper = (lax.broadcasted_iota(jnp.int32, (BLK, BLK), 1) >
             lax.broadcasted_iota(jnp.int32, (BLK, BLK), 0))
    T = []
    U_row = jnp.zeros((1, N), f32)
    for ib in range(NB):
        ai = acs[ib]
        dF = jnp.where((rows[ib] > ai) & upper, f32(1), f32(0))
        t = lax.dot_general(dF, ones_col, (((1,), (0,)), ((), ())),
                            preferred_element_type=f32)
        u = lax.dot_general(ones_col, dF, (((0,), (0,)), ((), ())),
                            preferred_element_type=f32)
        parts = [u]
        if ib + 1 < NB:
            ar = a_row[:, (ib + 1) * BLK:]
            wu = N - (ib + 1) * BLK
            xF = jnp.where(ar > ai, f32(1), f32(0))
            t = t + lax.dot_general(xF, jnp.ones((wu, 1), f32),
                                    (((1,), (0,)), ((), ())),
                                    preferred_element_type=f32)
            parts.append(lax.dot_general(ones_col, xF, (((0,), (0,)), ((), ())),
                                         preferred_element_type=f32))
        if ib > 0:
            parts.insert(0, jnp.zeros((1, ib * BLK), f32))
        U_row = U_row + jnp.concatenate(parts, axis=1)
        T.append(t)

    U_col = jnp.transpose(U_row)
    iif = lax.broadcasted_iota(jnp.int32, (N, 1), 0).astype(f32)
    rank_full = jnp.concatenate(T, axis=0) + iif - U_col

    p_rowF = lax.broadcasted_iota(jnp.int32, (1, K), 1).astype(f32)
    jidsF = lax.broadcasted_iota(jnp.int32, (N, 1), 0).astype(f32)
    picked = jnp.where(rank_full == p_rowF, jidsF, f32(0))
    idxF = jnp.sum(picked, axis=0, keepdims=True)
    idx_ref[0] = idxF.astype(jnp.int32)


_topk = pl.pallas_call(
    _topk_body,
    grid=(B,),
    in_specs=[pl.BlockSpec((1, N, C), lambda i: (i, 0, 0)),
              pl.BlockSpec((C, 1), lambda i: (0, 0)),
              pl.BlockSpec((1, 1), lambda i: (0, 0))],
    out_specs=pl.BlockSpec((1, 1, K), lambda i: (i, 0, 0)),
    out_shape=jax.ShapeDtypeStruct((B, 1, K), jnp.int32),
)


CH = 32
NCH = RPW // CH


NH = RPW // 2


def _sc_gather_body(idx_hbm, nodes_hbm, adj_hbm, nodes_out, adj_out,
                    idxv, gidx, nidx, nbuf, abuf0, abuf1,
                    nsem, nwsem, asem0, asem1):
    wid = lax.axis_index("s") * 2 + lax.axis_index("c")
    base = wid * RPW
    pltpu.sync_copy(idx_hbm.at[pl.ds(base, RPW)], idxv)
    g = wid // (K // RPW)
    per = CH // 16
    nper = NH // 16
    for i in range(RPW // 16):
        v = idxv[pl.ds(i * 16, 16)] + g * N
        gidx[i // per, pl.ds((i % per) * 16, 16)] = v
        nidx[i // nper, pl.ds((i % nper) * 16, 16)] = v
    abufs = (abuf0, abuf1)
    asems = (asem0, asem1)
    ncp = pltpu.async_copy(nodes_hbm.at[nidx.at[0]], nbuf, nsem)
    nwr = None
    acps = {0: pltpu.async_copy(adj_hbm.at[gidx.at[0], pl.ds(0, K)],
                                abuf0, asem0)}
    for c in range(NCH):
        if c + 1 < NCH:
            acps[c + 1] = pltpu.async_copy(
                adj_hbm.at[gidx.at[c + 1], pl.ds(0, K)],
                abufs[(c + 1) % 2], asems[(c + 1) % 2])
        if c == 2:
            ncp.wait()
            nwr = pltpu.async_copy(nbuf, nodes_out.at[pl.ds(base, NH)], nwsem)
        elif c == 4:
            nwr.wait()
            ncp = pltpu.async_copy(nodes_hbm.at[nidx.at[1]], nbuf, nsem)
        elif c == 6:
            ncp.wait()
            nwr = pltpu.async_copy(nbuf, nodes_out.at[pl.ds(base + NH, NH)],
                                   nwsem)
        acps[c].wait()
        pltpu.sync_copy(abufs[c % 2], adj_out.at[pl.ds(base + c * CH, CH)])
    nwr.wait()


def _sc_gather(idx_flat, nodes_flat, adj_flat):
    mesh = plsc.VectorSubcoreMesh(core_axis_name="c", subcore_axis_name="s")
    kern = pl.kernel(
        _sc_gather_body,
        out_type=[jax.ShapeDtypeStruct((B * K, C), jnp.float32),
                  jax.ShapeDtypeStruct((B * K, K), jnp.float32)],
        mesh=mesh,
        scratch_types=[pltpu.VMEM((RPW,), jnp.int32),
                       pltpu.VMEM((NCH, CH), jnp.int32),
                       pltpu.VMEM((2, NH), jnp.int32),
                       pltpu.VMEM((NH, C), jnp.float32),
                       pltpu.VMEM((CH, K), jnp.float32),
                       pltpu.VMEM((CH, K), jnp.float32),
                       pltpu.SemaphoreType.DMA,
                       pltpu.SemaphoreType.DMA,
                       pltpu.SemaphoreType.DMA,
                       pltpu.SemaphoreType.DMA],
    )
    return kern(idx_flat, nodes_flat, adj_flat)


def kernel(nodes, adj_mat, W, b):
    idx3 = _topk(nodes, W, b.reshape(1, 1))
    idx_flat = idx3.reshape(B * K)
    nodes_flat = nodes.reshape(B * N, C)
    adj_flat = adj_mat.reshape(B * N, 2 * K)
    nodes_out, adj_out = _sc_gather(idx_flat, nodes_flat, adj_flat)
    return nodes_out.reshape(B, K, C), adj_out.reshape(B, K, K)


# speedup vs baseline: 1.0477x; 1.0286x over previous
import jax
import jax.numpy as jnp
from jax import lax
from jax.experimental import pallas as pl
from jax.experimental.pallas import tpu as pltpu
from jax.experimental.pallas import tpu_sc as plsc

B, N, C, K = 8, 2048, 256, 1024
BLK = 256
NW = 32
RPW = (B * K) // NW


def _topk_body(nodes_ref, w_ref, b_ref, idx_ref):
    f32 = jnp.float32
    nodes = nodes_ref[0]
    w = w_ref[...]
    bias = b_ref[0, 0]
    a_col = jnp.dot(nodes, w, preferred_element_type=f32) + bias

    NB = N // BLK
    a_row = jnp.transpose(a_col)
    acs = [a_col[ib * BLK:(ib + 1) * BLK, :] for ib in range(NB)]
    rows = [a_row[:, ib * BLK:(ib + 1) * BLK] for ib in range(NB)]

    ones_col = jnp.ones((BLK, 1), f32)
    upper = (lax.broadcasted_iota(jnp.int32, (BLK, BLK), 1) >
             lax.broadcasted_iota(jnp.int32, (BLK, BLK), 0))
    T = []
    U_row = jnp.zeros((1, N), f32)
    for ib in range(NB):
        ai = acs[ib]
        dF = jnp.where((rows[ib] > ai) & upper, f32(1), f32(0))
        t = lax.dot_general(dF, ones_col, (((1,), (0,)), ((), ())),
                            preferred_element_type=f32)
        u = lax.dot_general(ones_col, dF, (((0,), (0,)), ((), ())),
                            preferred_element_type=f32)
        parts = [u]
        if ib + 1 < NB:
            ar = a_row[:, (ib + 1) * BLK:]
            wu = N - (ib + 1) * BLK
            xF = jnp.where(ar > ai, f32(1), f32(0))
            t = t + lax.dot_general(xF, jnp.ones((wu, 1), f32),
                                    (((1,), (0,)), ((), ())),
                                    preferred_element_type=f32)
            parts.append(lax.dot_general(ones_col, xF, (((0,), (0,)), ((), ())),
                                         preferred_element_type=f32))
        if ib > 0:
            parts.insert(0, jnp.zeros((1, ib * BLK), f32))
        U_row = U_row + jnp.concatenate(parts, axis=1)
        T.append(t)

    U_col = jnp.transpose(U_row)
    iif = lax.broadcasted_iota(jnp.int32, (N, 1), 0).astype(f32)
    rank_full = jnp.concatenate(T, axis=0) + iif - U_col

    p_rowF = lax.broadcasted_iota(jnp.int32, (1, K), 1).astype(f32)
    jidsF = lax.broadcasted_iota(jnp.int32, (N, 1), 0).astype(f32)
    picked = jnp.where(rank_full == p_rowF, jidsF, f32(0))
    idxF = jnp.sum(picked, axis=0, keepdims=True)
    idx_ref[0] = idxF.astype(jnp.int32)


_topk = pl.pallas_call(
    _topk_body,
    grid=(B,),
    in_specs=[pl.BlockSpec((1, N, C), lambda i: (i, 0, 0)),
              pl.BlockSpec((C, 1), lambda i: (0, 0)),
              pl.BlockSpec((1, 1), lambda i: (0, 0))],
    out_specs=pl.BlockSpec((1, 1, K), lambda i: (i, 0, 0)),
    out_shape=jax.ShapeDtypeStruct((B, 1, K), jnp.int32),
)


CH = 32
NCH = RPW // CH


NH = RPW // 2


def _sc_gather_body(idx_hbm, nodes_hbm, adj_hbm, nodes_out, adj_out,
                    idxv, gidx, nidx, nbuf, abuf0, abuf1,
                    nsem, nwsem, asem0, asem1):
    wid = lax.axis_index("s") * 2 + lax.axis_index("c")
    base = wid * RPW
    pltpu.sync_copy(idx_hbm.at[pl.ds(base, RPW)], idxv)
    g = wid // (K // RPW)
    per = CH // 16
    nper = NH // 16
    for i in range(RPW // 16):
        v = idxv[pl.ds(i * 16, 16)] + g * N
        gidx[i // per, pl.ds((i % per) * 16, 16)] = v
        nidx[i // nper, pl.ds((i % nper) * 16, 16)] = v
    abufs = (abuf0, abuf1)
    asems = (asem0, asem1)
    ncp = pltpu.async_copy(nodes_hbm.at[nidx.at[0]], nbuf, nsem)
    nwr = None
    acps = {0: pltpu.async_copy(adj_hbm.at[gidx.at[0], pl.ds(0, K)],
                                abuf0, asem0)}
    for c in range(NCH):
        if c + 1 < NCH:
            acps[c + 1] = pltpu.async_copy(
                adj_hbm.at[gidx.at[c + 1], pl.ds(0, K)],
                abufs[(c + 1) % 2], asems[(c + 1) % 2])
        if c == 2:
            ncp.wait()
            nwr = pltpu.async_copy(nbuf, nodes_out.at[pl.ds(base, NH)], nwsem)
        elif c == 4:
            nwr.wait()
            ncp = pltpu.async_copy(nodes_hbm.at[nidx.at[1]], nbuf, nsem)
        elif c == 6:
            ncp.wait()
            nwr = pltpu.async_copy(nbuf, nodes_out.at[pl.ds(base + NH, NH)],
                                   nwsem)
        acps[c].wait()
        pltpu.sync_copy(abufs[c % 2], adj_out.at[pl.ds(base + c * CH, CH)])
    nwr.wait()


def _sc_gather(idx_flat, nodes_flat, adj_flat):
    mesh = plsc.VectorSubcoreMesh(core_axis_name="c", subcore_axis_name="s")
    kern = pl.kernel(
        _sc_gather_body,
        out_type=[jax.ShapeDtypeStruct((B * K, C), jnp.float32),
                  jax.ShapeDtypeStruct((B * K, K), jnp.float32)],
        mesh=mesh,
        scratch_types=[pltpu.VMEM((RPW,), jnp.int32),
                       pltpu.VMEM((NCH, CH), jnp.int32),
                       pltpu.VMEM((2, NH), jnp.int32),
                       pltpu.VMEM((NH, C), jnp.float32),
                       pltpu.VMEM((CH, K), jnp.float32),
                       pltpu.VMEM((CH, K), jnp.float32),
                       pltpu.SemaphoreType.DMA,
                       pltpu.SemaphoreType.DMA,
                       pltpu.SemaphoreType.DMA,
                       pltpu.SemaphoreType.DMA],
    )
    return kern(idx_flat, nodes_flat, adj_flat)


def kernel(nodes, adj_mat, W, b):
    idx3 = _topk(nodes, W, b.reshape(1, 1))
    idx_flat = idx3.reshape(B * K)
    nodes_flat = nodes.reshape(B * N, C)
    adj_flat = adj_mat.reshape(B * N, 2 * K)
    nodes_out, adj_out = _sc_gather(idx_flat, nodes_flat, adj_flat)
    return nodes_out.reshape(B, K, C), adj_out.reshape(B, K, K)


# speedup vs baseline: 1.0502x; 1.0024x over previous
import jax
import jax.numpy as jnp
from jax import lax
from jax.experimental import pallas as pl
from jax.experimental.pallas import tpu as pltpu
from jax.experimental.pallas import tpu_sc as plsc

B, N, C, K = 8, 2048, 256, 1024
BLK = 256
NW = 32
RPW = (B * K) // NW


def _topk_body(nodes_ref, w_ref, b_ref, idx_ref):
    f32 = jnp.float32
    nodes = nodes_ref[0]
    w = w_ref[...]
    bias = b_ref[0, 0]
    a_col = jnp.dot(nodes, w, preferred_element_type=f32) + bias

    NB = N // BLK
    a_row = jnp.transpose(a_col)
    acs = [a_col[ib * BLK:(ib + 1) * BLK, :] for ib in range(NB)]
    rows = [a_row[:, ib * BLK:(ib + 1) * BLK] for ib in range(NB)]

    ones_col = jnp.ones((BLK, 1), f32)
    upper = (lax.broadcasted_iota(jnp.int32, (BLK, BLK), 1) >
             lax.broadcasted_iota(jnp.int32, (BLK, BLK), 0))
    T = []
    U_row = jnp.zeros((1, N), f32)
    for ib in range(NB):
        ai = acs[ib]
        dF = jnp.where((rows[ib] > ai) & upper, f32(1), f32(0))
        t = lax.dot_general(dF, ones_col, (((1,), (0,)), ((), ())),
                            preferred_element_type=f32)
        u = lax.dot_general(ones_col, dF, (((0,), (0,)), ((), ())),
                            preferred_element_type=f32)
        parts = [u]
        if ib + 1 < NB:
            ar = a_row[:, (ib + 1) * BLK:]
            wu = N - (ib + 1) * BLK
            xF = jnp.where(ar > ai, f32(1), f32(0))
            t = t + lax.dot_general(xF, jnp.ones((wu, 1), f32),
                                    (((1,), (0,)), ((), ())),
                                    preferred_element_type=f32)
            parts.append(lax.dot_general(ones_col, xF, (((0,), (0,)), ((), ())),
                                         preferred_element_type=f32))
        if ib > 0:
            parts.insert(0, jnp.zeros((1, ib * BLK), f32))
        U_row = U_row + jnp.concatenate(parts, axis=1)
        T.append(t)

    U_col = jnp.transpose(U_row)
    iif = lax.broadcasted_iota(jnp.int32, (N, 1), 0).astype(f32)
    rank_full = jnp.concatenate(T, axis=0) + iif - U_col

    p_rowF = lax.broadcasted_iota(jnp.int32, (1, K), 1).astype(f32)
    jidsF = lax.broadcasted_iota(jnp.int32, (N, 1), 0).astype(f32)
    picked = jnp.where(rank_full == p_rowF, jidsF, f32(0))
    idxF = jnp.sum(picked, axis=0, keepdims=True)
    idx_ref[0] = idxF.astype(jnp.int32)


_topk = pl.pallas_call(
    _topk_body,
    grid=(B,),
    in_specs=[pl.BlockSpec((1, N, C), lambda i: (i, 0, 0)),
              pl.BlockSpec((C, 1), lambda i: (0, 0)),
              pl.BlockSpec((1, 1), lambda i: (0, 0))],
    out_specs=pl.BlockSpec((1, 1, K), lambda i: (i, 0, 0)),
    out_shape=jax.ShapeDtypeStruct((B, 1, K), jnp.int32),
)


CH = 16
NCH = RPW // CH


NH = RPW // 2


def _sc_gather_body(idx_hbm, nodes_hbm, adj_hbm, nodes_out, adj_out,
                    idxv, gidx, nidx, nbuf, abuf0, abuf1,
                    nsem, nwsem, asem0, asem1):
    wid = lax.axis_index("s") * 2 + lax.axis_index("c")
    base = wid * RPW
    pltpu.sync_copy(idx_hbm.at[pl.ds(base, RPW)], idxv)
    g = wid // (K // RPW)
    per = CH // 16
    nper = NH // 16
    for i in range(RPW // 16):
        v = idxv[pl.ds(i * 16, 16)] + g * N
        gidx[i // per, pl.ds((i % per) * 16, 16)] = v
        nidx[i // nper, pl.ds((i % nper) * 16, 16)] = v
    abufs = (abuf0, abuf1)
    asems = (asem0, asem1)
    ncp = pltpu.async_copy(nodes_hbm.at[nidx.at[0]], nbuf, nsem)
    nwr = None
    acps = {0: pltpu.async_copy(adj_hbm.at[gidx.at[0], pl.ds(0, K)],
                                abuf0, asem0)}
    for c in range(NCH):
        if c + 1 < NCH:
            acps[c + 1] = pltpu.async_copy(
                adj_hbm.at[gidx.at[c + 1], pl.ds(0, K)],
                abufs[(c + 1) % 2], asems[(c + 1) % 2])
        if c == 2:
            ncp.wait()
            nwr = pltpu.async_copy(nbuf, nodes_out.at[pl.ds(base, NH)], nwsem)
        elif c == 4:
            nwr.wait()
            ncp = pltpu.async_copy(nodes_hbm.at[nidx.at[1]], nbuf, nsem)
        elif c == 6:
            ncp.wait()
            nwr = pltpu.async_copy(nbuf, nodes_out.at[pl.ds(base + NH, NH)],
                                   nwsem)
        acps[c].wait()
        pltpu.sync_copy(abufs[c % 2], adj_out.at[pl.ds(base + c * CH, CH)])
    nwr.wait()


def _sc_gather(idx_flat, nodes_flat, adj_flat):
    mesh = plsc.VectorSubcoreMesh(core_axis_name="c", subcore_axis_name="s")
    kern = pl.kernel(
        _sc_gather_body,
        out_type=[jax.ShapeDtypeStruct((B * K, C), jnp.float32),
                  jax.ShapeDtypeStruct((B * K, K), jnp.float32)],
        mesh=mesh,
        scratch_types=[pltpu.VMEM((RPW,), jnp.int32),
                       pltpu.VMEM((NCH, CH), jnp.int32),
                       pltpu.VMEM((2, NH), jnp.int32),
                       pltpu.VMEM((NH, C), jnp.float32),
                       pltpu.VMEM((CH, K), jnp.float32),
                       pltpu.VMEM((CH, K), jnp.float32),
                       pltpu.SemaphoreType.DMA,
                       pltpu.SemaphoreType.DMA,
                       pltpu.SemaphoreType.DMA,
                       pltpu.SemaphoreType.DMA],
    )
    return kern(idx_flat, nodes_flat, adj_flat)


def kernel(nodes, adj_mat, W, b):
    idx3 = _topk(nodes, W, b.reshape(1, 1))
    idx_flat = idx3.reshape(B * K)
    nodes_flat = nodes.reshape(B * N, C)
    adj_flat = adj_mat.reshape(B * N, 2 * K)
    nodes_out, adj_out = _sc_gather(idx_flat, nodes_flat, adj_flat)
    return nodes_out.reshape(B, K, C), adj_out.reshape(B, K, K)
